# Initial kernel scaffold; baseline (speedup 1.0000x reference)
import jax
import jax.numpy as jnp
from jax.experimental import pallas as pl


def kernel(seq, mods, aa_table, mod_table, pe):
    raise NotImplementedError("write your pallas kernel here")


# speedup vs baseline: 8.8919x; 8.8919x over previous
import functools
import math

import jax
import jax.numpy as jnp
from jax import lax
from jax.experimental import pallas as pl
from jax.experimental.pallas import tpu as pltpu
from jax.experimental.pallas import tpu_sc as plsc

D_MODEL = 128
AA_VOCAB = 24
MOD_PAD = 16
COMBO = AA_VOCAB * MOD_PAD
SCALE = math.sqrt(24.0)

NUM_CORES = 2
NUM_SUBCORES = 16
NUM_WORKERS = NUM_CORES * NUM_SUBCORES


def _prep_body(aa_ref, modp_ref, pe_ref, seq_ref, mods_ref, fused_ref, ci_ref):
    base = (aa_ref[...][:, None, :] + modp_ref[...][None, :, :]) * SCALE
    base = base.reshape(COMBO, D_MODEL)
    fused_ref[...] = base[None, :, :] + pe_ref[...][:, None, :]

    nw, L, bpw = ci_ref.shape
    x = seq_ref[...] * MOD_PAD + mods_ref[...]
    x = x.reshape(nw, bpw, L)
    x = jnp.swapaxes(x, 1, 2)
    ci_ref[...] = x + lax.broadcasted_iota(jnp.int32, (nw, L, bpw), 1) * COMBO


def _tc_prep(aa_table, mod_table, pe50, seq, mods, L, B):
    modp = jnp.pad(mod_table, ((0, MOD_PAD - mod_table.shape[0]), (0, 0)))
    bpw = B // NUM_WORKERS
    fused, ci = pl.pallas_call(
        _prep_body,
        out_shape=[
            jax.ShapeDtypeStruct((L, COMBO, D_MODEL), jnp.float32),
            jax.ShapeDtypeStruct((NUM_WORKERS, L, bpw), jnp.int32),
        ],
    )(aa_table, modp, pe50, seq, mods)
    return fused.reshape(L * COMBO, D_MODEL), ci


def _sc_lookup(fused, ci, B, L):
    bpw = B // NUM_WORKERS
    mesh = plsc.VectorSubcoreMesh(core_axis_name="c", subcore_axis_name="s")

    @functools.partial(
        pl.kernel,
        mesh=mesh,
        out_type=jax.ShapeDtypeStruct((L, B, D_MODEL), jnp.float32),
        scratch_types=[
            pltpu.VMEM((L, bpw), jnp.int32),
            pltpu.VMEM((bpw, D_MODEL), jnp.float32),
            pltpu.SemaphoreType.DMA,
        ],
    )
    def body(fused_hbm, ci_hbm, out_hbm, ci_v, buf_v, gsem):
        wid = lax.axis_index("s") * NUM_CORES + lax.axis_index("c")
        b0 = wid * bpw
        pltpu.sync_copy(ci_hbm.at[wid], ci_v)

        def do_l(l, carry):
            pltpu.async_copy(fused_hbm.at[ci_v.at[l]], buf_v, gsem).wait()
            pltpu.sync_copy(buf_v, out_hbm.at[l, pl.ds(b0, bpw)])
            return carry

        lax.fori_loop(0, L, do_l, 0)

    return body(fused, ci)


def kernel(seq, mods, aa_table, mod_table, pe):
    seq = seq.astype(jnp.int32)
    mods = mods.astype(jnp.int32)
    L = seq.shape[1]
    B = seq.shape[0]
    pad = L - mods.shape[1]
    if pad:
        mods = jnp.pad(mods, ((0, 0), (0, pad)))
    pe50 = pe[:L, 0, :]
    fused, ci = _tc_prep(aa_table, mod_table, pe50, seq, mods, L, B)
    return _sc_lookup(fused, ci, B, L)


# speedup vs baseline: 10.1562x; 1.1422x over previous
import functools
import math

import jax
import jax.numpy as jnp
from jax import lax
from jax.experimental import pallas as pl
from jax.experimental.pallas import tpu as pltpu
from jax.experimental.pallas import tpu_sc as plsc

D_MODEL = 128
AA_VOCAB = 24
MOD_PAD = 16
COMBO = AA_VOCAB * MOD_PAD
SCALE = math.sqrt(24.0)

NUM_CORES = 2
NUM_SUBCORES = 16
NUM_WORKERS = NUM_CORES * NUM_SUBCORES


def _prep_body(aa_ref, modp_ref, pe_ref, seq_ref, mods_ref, fused_ref, ci_ref):
    base = (aa_ref[...][:, None, :] + modp_ref[...][None, :, :]) * SCALE
    base = base.reshape(COMBO, D_MODEL)
    fused_ref[...] = base[None, :, :] + pe_ref[...][:, None, :]

    nw, L, bpw = ci_ref.shape
    x = seq_ref[...] * MOD_PAD + mods_ref[...]
    x = x.reshape(nw, bpw, L)
    x = jnp.swapaxes(x, 1, 2)
    ci_ref[...] = x + lax.broadcasted_iota(jnp.int32, (nw, L, bpw), 1) * COMBO


def _tc_prep(aa_table, mod_table, pe50, seq, mods, L, B):
    modp = jnp.pad(mod_table, ((0, MOD_PAD - mod_table.shape[0]), (0, 0)))
    bpw = B // NUM_WORKERS
    fused, ci = pl.pallas_call(
        _prep_body,
        out_shape=[
            jax.ShapeDtypeStruct((L, COMBO, D_MODEL), jnp.float32),
            jax.ShapeDtypeStruct((NUM_WORKERS, L, bpw), jnp.int32),
        ],
    )(aa_table, modp, pe50, seq, mods)
    return fused.reshape(L * COMBO, D_MODEL), ci


def _sc_lookup(fused, ci, B, L):
    bpw = B // NUM_WORKERS
    mesh = plsc.VectorSubcoreMesh(core_axis_name="c", subcore_axis_name="s")

    @functools.partial(
        pl.kernel,
        mesh=mesh,
        out_type=jax.ShapeDtypeStruct((L, B, D_MODEL), jnp.float32),
        scratch_types=[
            pltpu.VMEM((L, bpw), jnp.int32),
            pltpu.VMEM((bpw, D_MODEL), jnp.float32),
            pltpu.VMEM((bpw, D_MODEL), jnp.float32),
            pltpu.SemaphoreType.DMA,
            pltpu.SemaphoreType.DMA,
            pltpu.SemaphoreType.DMA,
            pltpu.SemaphoreType.DMA,
        ],
    )
    def body(fused_hbm, ci_hbm, out_hbm, ci_v, buf_a, buf_b, gsem_a, gsem_b,
             ssem_a, ssem_b):
        wid = lax.axis_index("s") * NUM_CORES + lax.axis_index("c")
        b0 = wid * bpw
        pltpu.sync_copy(ci_hbm.at[wid], ci_v)

        def do_pair(i, carry):
            l0 = 2 * i
            l1 = l0 + 1

            @pl.when(i > 0)
            def _():
                pltpu.make_async_copy(buf_a, out_hbm.at[l0, pl.ds(b0, bpw)],
                                      ssem_a).wait()

            pltpu.async_copy(fused_hbm.at[ci_v.at[l0]], buf_a, gsem_a).wait()
            pltpu.async_copy(buf_a, out_hbm.at[l0, pl.ds(b0, bpw)], ssem_a)

            @pl.when(i > 0)
            def _():
                pltpu.make_async_copy(buf_b, out_hbm.at[l1, pl.ds(b0, bpw)],
                                      ssem_b).wait()

            pltpu.async_copy(fused_hbm.at[ci_v.at[l1]], buf_b, gsem_b).wait()
            pltpu.async_copy(buf_b, out_hbm.at[l1, pl.ds(b0, bpw)], ssem_b)
            return carry

        lax.fori_loop(0, L // 2, do_pair, 0)
        pltpu.make_async_copy(buf_a, out_hbm.at[0, pl.ds(b0, bpw)], ssem_a).wait()
        pltpu.make_async_copy(buf_b, out_hbm.at[0, pl.ds(b0, bpw)], ssem_b).wait()

    return body(fused, ci)


def kernel(seq, mods, aa_table, mod_table, pe):
    seq = seq.astype(jnp.int32)
    mods = mods.astype(jnp.int32)
    L = seq.shape[1]
    B = seq.shape[0]
    pad = L - mods.shape[1]
    if pad:
        mods = jnp.pad(mods, ((0, 0), (0, pad)))
    pe50 = pe[:L, 0, :]
    fused, ci = _tc_prep(aa_table, mod_table, pe50, seq, mods, L, B)
    return _sc_lookup(fused, ci, B, L)
